# Initial kernel scaffold; baseline (speedup 1.0000x reference)
import functools

import jax
import jax.numpy as jnp
from jax import lax
from jax.experimental import pallas as pl
from jax.experimental.pallas import tpu as pltpu

LOG2N = 15
N = 1 << LOG2N
SUB = 256
LANE = 128


def _partner(x, j):
    if j < SUB:
        if j >= 8:
            g = SUB // (2 * j)
            v = x.reshape(x.shape[0], g, 2, j, LANE)
            return jnp.flip(v, axis=2).reshape(x.shape)
        up = pltpu.roll(x, -j, 1)
        dn = pltpu.roll(x, j, 1)
        sub = lax.broadcasted_iota(jnp.int32, (1, SUB, 1), 1)
        return jnp.where((sub & j) == 0, up, dn)
    s = j // SUB
    up = pltpu.roll(x, -s, 2)
    dn = pltpu.roll(x, s, 2)
    ln = lax.broadcasted_iota(jnp.int32, (1, 1, LANE), 2)
    return jnp.where((ln & s) == 0, up, dn)


def _sort2(a, b, idx):
    for km in range(1, LOG2N + 1):
        k = 1 << km
        for jm in range(km - 1, -1, -1):
            j = 1 << jm
            if k == N:
                takemin = (idx & j) == 0
            else:
                t = idx & (j | k)
                takemin = (t == 0) | (t == (j | k))
            pa = _partner(a, j)
            pb = _partner(b, j)
            a = jnp.where(takemin, jnp.minimum(a, pa), jnp.maximum(a, pa))
            b = jnp.where(takemin, jnp.minimum(b, pb), jnp.maximum(b, pb))
    return a, b


def _loss_kernel(t_ref, p_ref, out_ref):
    idx = (lax.broadcasted_iota(jnp.int32, (1, SUB, LANE), 2) * SUB
           + lax.broadcasted_iota(jnp.int32, (1, SUB, LANE), 1))
    t = t_ref[...]
    p = p_ref[...]
    d0 = t - p
    s_plain = jnp.sum(d0 * d0)
    ts, ps = _sort2(t, p, idx)
    d1 = ts - ps
    s_cdf = jnp.sum(d1 * d1)
    out_ref[0, 0] = s_plain + s_cdf


@jax.jit
def kernel(predictions, targets):
    rows, n = predictions.shape
    assert n == N
    t3 = targets.reshape(rows, SUB, LANE)
    p3 = predictions.reshape(rows, SUB, LANE)
    partials = pl.pallas_call(
        _loss_kernel,
        grid=(rows,),
        in_specs=[
            pl.BlockSpec((1, SUB, LANE), lambda i: (i, 0, 0)),
            pl.BlockSpec((1, SUB, LANE), lambda i: (i, 0, 0)),
        ],
        out_specs=pl.BlockSpec((1, 1), lambda i: (i, 0)),
        out_shape=jax.ShapeDtypeStruct((rows, 1), jnp.float32),
        compiler_params=pltpu.CompilerParams(
            dimension_semantics=("parallel",),
        ),
    )(t3, p3)
    total = jnp.sum(partials)
    return total / (rows * N)


# speedup vs baseline: 2.8304x; 2.8304x over previous
import functools

import jax
import jax.numpy as jnp
from jax import lax
from jax.experimental import pallas as pl
from jax.experimental.pallas import tpu as pltpu

LOG2N = 15
N = 1 << LOG2N
SUB = 256
LANE = 128


def _partner(x, j):
    if j < SUB:
        if j >= 8:
            g = SUB // (2 * j)
            v = x.reshape(x.shape[0], g, 2, j, LANE)
            sw = jnp.concatenate([v[:, :, 1:2], v[:, :, 0:1]], axis=2)
            return sw.reshape(x.shape)
        up = pltpu.roll(x, SUB - j, 1)
        dn = pltpu.roll(x, j, 1)
        sub = lax.broadcasted_iota(jnp.int32, (1, SUB, 1), 1)
        return jnp.where((sub & j) == 0, up, dn)
    s = j // SUB
    up = pltpu.roll(x, LANE - s, 2)
    dn = pltpu.roll(x, s, 2)
    ln = lax.broadcasted_iota(jnp.int32, (1, 1, LANE), 2)
    return jnp.where((ln & s) == 0, up, dn)


def _sort2(a, b, idx):
    for km in range(1, LOG2N + 1):
        k = 1 << km
        if k == N:
            y = idx
        else:
            y = jnp.where((idx & k) == 0, idx, ~idx)
        for jm in range(km - 1, -1, -1):
            j = 1 << jm
            takemin = (y & j) == 0
            pa = _partner(a, j)
            pb = _partner(b, j)
            a = jnp.where(takemin, jnp.minimum(a, pa), jnp.maximum(a, pa))
            b = jnp.where(takemin, jnp.minimum(b, pb), jnp.maximum(b, pb))
    return a, b


def _loss_kernel(t_ref, p_ref, out_ref):
    idx = (lax.broadcasted_iota(jnp.int32, (1, SUB, LANE), 2) * SUB
           + lax.broadcasted_iota(jnp.int32, (1, SUB, LANE), 1))
    t = t_ref[...]
    p = p_ref[...]
    d0 = t - p
    s_plain = jnp.sum(d0 * d0)
    ts, ps = _sort2(t, p, idx)
    d1 = ts - ps
    s_cdf = jnp.sum(d1 * d1)
    out_ref[...] = (s_plain + s_cdf).reshape(1, 1, 1)


@jax.jit
def kernel(predictions, targets):
    rows, n = predictions.shape
    assert n == N
    t3 = targets.reshape(rows, SUB, LANE)
    p3 = predictions.reshape(rows, SUB, LANE)
    partials = pl.pallas_call(
        _loss_kernel,
        grid=(rows,),
        in_specs=[
            pl.BlockSpec((1, SUB, LANE), lambda i: (i, 0, 0)),
            pl.BlockSpec((1, SUB, LANE), lambda i: (i, 0, 0)),
        ],
        out_specs=pl.BlockSpec((1, 1, 1), lambda i: (i, 0, 0)),
        out_shape=jax.ShapeDtypeStruct((rows, 1, 1), jnp.float32),
        compiler_params=pltpu.CompilerParams(
            dimension_semantics=("parallel",),
        ),
    )(t3, p3)
    total = jnp.sum(partials)
    return total / (rows * N)


# speedup vs baseline: 3.4852x; 1.2313x over previous
import functools

import jax
import jax.numpy as jnp
from jax import lax
from jax.experimental import pallas as pl
from jax.experimental.pallas import tpu as pltpu

LOG2N = 15
N = 1 << LOG2N
SUB = 256
LANE = 128


def _cmpx(x, j, takemin):
    if 8 <= j < SUB:
        g = SUB // (2 * j)
        v = x.reshape(x.shape[0], g, 2, j, LANE)
        a = v[:, :, 0]
        b = v[:, :, 1]
        lo = jnp.minimum(a, b)
        hi = jnp.maximum(a, b)
        out = jnp.concatenate([lo[:, :, None], hi[:, :, None]], axis=2)
        return out.reshape(x.shape[0], SUB, LANE)
    if j < SUB:
        up = pltpu.roll(x, SUB - j, 1)
        dn = pltpu.roll(x, j, 1)
    else:
        s = j // SUB
        up = pltpu.roll(x, LANE - s, 2)
        dn = pltpu.roll(x, s, 2)
    p = jnp.where(takemin, up, dn)
    return jnp.where(takemin, jnp.minimum(x, p), jnp.maximum(x, p))


def _stage_mask(j):
    if j < SUB:
        bit = lax.broadcasted_iota(jnp.int32, (1, SUB, 1), 1) & j
    else:
        bit = lax.broadcasted_iota(jnp.int32, (1, 1, LANE), 2) & (j // SUB)
    return bit == 0


def _sort2_asc(a, b, idx):
    asc2 = (idx & 2) == 0
    a = jnp.where(asc2, a, -a)
    b = jnp.where(asc2, b, -b)
    for km in range(1, LOG2N + 1):
        k = 1 << km
        if km > 1:
            prev = k >> 1
            flip = ((idx & prev) == 0) != ((idx & k) == 0)
            a = jnp.where(flip, -a, a)
            b = jnp.where(flip, -b, b)
        for jm in range(km - 1, -1, -1):
            j = 1 << jm
            takemin = None if 8 <= j < SUB else _stage_mask(j)
            a = _cmpx(a, j, takemin)
            b = _cmpx(b, j, takemin)
    return a, b


def _loss_kernel(t_ref, p_ref, out_ref):
    idx = (lax.broadcasted_iota(jnp.int32, (1, SUB, LANE), 2) * SUB
           + lax.broadcasted_iota(jnp.int32, (1, SUB, LANE), 1))
    t = t_ref[...]
    p = p_ref[...]
    d0 = t - p
    s_plain = jnp.sum(d0 * d0)
    ts, ps = _sort2_asc(t, p, idx)
    d1 = ts - ps
    s_cdf = jnp.sum(d1 * d1)
    out_ref[...] = (s_plain + s_cdf).reshape(1, 1, 1)


@jax.jit
def kernel(predictions, targets):
    rows, n = predictions.shape
    assert n == N
    t3 = targets.reshape(rows, SUB, LANE)
    p3 = predictions.reshape(rows, SUB, LANE)
    blk = 1
    partials = pl.pallas_call(
        _loss_kernel,
        grid=(rows // blk,),
        in_specs=[
            pl.BlockSpec((blk, SUB, LANE), lambda i: (i, 0, 0)),
            pl.BlockSpec((blk, SUB, LANE), lambda i: (i, 0, 0)),
        ],
        out_specs=pl.BlockSpec((1, 1, 1), lambda i: (i, 0, 0)),
        out_shape=jax.ShapeDtypeStruct((rows // blk, 1, 1), jnp.float32),
        compiler_params=pltpu.CompilerParams(
            dimension_semantics=("parallel",),
        ),
    )(t3, p3)
    total = jnp.sum(partials)
    return total / (rows * N)


# speedup vs baseline: 4.3070x; 1.2358x over previous
import functools

import jax
import jax.numpy as jnp
from jax import lax
from jax.experimental import pallas as pl
from jax.experimental.pallas import tpu as pltpu

LOG2N = 15
N = 1 << LOG2N
SUB = 256
LANE = 128


def _cmpx(x, j, takemin):
    if j < 32:
        cj = 8 * j
        g = SUB // (2 * cj)
        v = x.reshape(x.shape[0], g, 2, cj, LANE)
        a = v[:, :, 0]
        b = v[:, :, 1]
        lo = jnp.minimum(a, b)
        hi = jnp.maximum(a, b)
        out = jnp.concatenate([lo[:, :, None], hi[:, :, None]], axis=2)
        return out.reshape(x.shape[0], SUB, LANE)
    if j < SUB:
        q = j >> 5
        up = pltpu.roll(x, SUB - q, 1)
        dn = pltpu.roll(x, q, 1)
    else:
        s = j >> 8
        up = pltpu.roll(x, LANE - s, 2)
        dn = pltpu.roll(x, s, 2)
    return jnp.where(takemin, jnp.minimum(x, up), jnp.maximum(x, dn))


def _stage_mask(j):
    if j < SUB:
        bit = lax.broadcasted_iota(jnp.int32, (1, SUB, 1), 1) & (j >> 5)
    else:
        bit = lax.broadcasted_iota(jnp.int32, (1, 1, LANE), 2) & (j >> 8)
    return bit == 0


def _sort2_asc(a, b, idx):
    asc2 = (idx & 2) == 0
    a = jnp.where(asc2, a, -a)
    b = jnp.where(asc2, b, -b)
    for km in range(1, LOG2N + 1):
        k = 1 << km
        if km > 1:
            prev = k >> 1
            flip = ((idx & prev) == 0) != ((idx & k) == 0)
            a = jnp.where(flip, -a, a)
            b = jnp.where(flip, -b, b)
        for jm in range(km - 1, -1, -1):
            j = 1 << jm
            takemin = None if j < 32 else _stage_mask(j)
            a = _cmpx(a, j, takemin)
            b = _cmpx(b, j, takemin)
    return a, b


def _loss_kernel(t_ref, p_ref, out_ref):
    r = lax.broadcasted_iota(jnp.int32, (1, SUB, LANE), 1)
    c = lax.broadcasted_iota(jnp.int32, (1, SUB, LANE), 2)
    idx = (r >> 3) | ((r & 7) << 5) | (c << 8)
    t = t_ref[...]
    p = p_ref[...]
    d0 = t - p
    s_plain = jnp.sum(d0 * d0)
    ts, ps = _sort2_asc(t, p, idx)
    d1 = ts - ps
    s_cdf = jnp.sum(d1 * d1)
    out_ref[...] = (s_plain + s_cdf).reshape(1, 1, 1)


@jax.jit
def kernel(predictions, targets):
    rows, n = predictions.shape
    assert n == N
    t3 = targets.reshape(rows, SUB, LANE)
    p3 = predictions.reshape(rows, SUB, LANE)
    partials = pl.pallas_call(
        _loss_kernel,
        grid=(rows,),
        in_specs=[
            pl.BlockSpec((1, SUB, LANE), lambda i: (i, 0, 0)),
            pl.BlockSpec((1, SUB, LANE), lambda i: (i, 0, 0)),
        ],
        out_specs=pl.BlockSpec((1, 1, 1), lambda i: (i, 0, 0)),
        out_shape=jax.ShapeDtypeStruct((rows, 1, 1), jnp.float32),
        compiler_params=pltpu.CompilerParams(
            dimension_semantics=("parallel",),
        ),
    )(t3, p3)
    total = jnp.sum(partials)
    return total / (rows * N)


# speedup vs baseline: 4.4678x; 1.0374x over previous
import functools

import jax
import jax.numpy as jnp
from jax import lax
from jax.experimental import pallas as pl
from jax.experimental.pallas import tpu as pltpu

LOG2N = 15
N = 1 << LOG2N
SUB = 256
LANE = 128


def _cmpx(x, j, takemin):
    if j < 32:
        cj = 8 * j
        g = SUB // (2 * cj)
        v = x.reshape(x.shape[0], g, 2, cj, LANE)
        a = v[:, :, 0]
        b = v[:, :, 1]
        lo = jnp.minimum(a, b)
        hi = jnp.maximum(a, b)
        out = jnp.concatenate([lo[:, :, None], hi[:, :, None]], axis=2)
        return out.reshape(x.shape[0], SUB, LANE)
    if j < SUB:
        q = j >> 5
        up = pltpu.roll(x, SUB - q, 1)
        dn = pltpu.roll(x, q, 1)
    else:
        s = j >> 8
        up = pltpu.roll(x, LANE - s, 2)
        dn = pltpu.roll(x, s, 2)
    return jnp.where(takemin, jnp.minimum(x, up), jnp.maximum(x, dn))


def _sort2_asc(a, b, idx):
    sub_iota = lax.broadcasted_iota(jnp.int32, (1, SUB, 1), 1)
    lane_iota = lax.broadcasted_iota(jnp.int32, (1, 1, LANE), 2)
    masks = {}
    for jm in range(5, LOG2N):
        j = 1 << jm
        if j < SUB:
            masks[j] = (sub_iota & (j >> 5)) == 0
        else:
            masks[j] = (lane_iota & (j >> 8)) == 0
    asc2 = (idx & 2) == 0
    a = jnp.where(asc2, a, -a)
    b = jnp.where(asc2, b, -b)
    for km in range(1, LOG2N + 1):
        k = 1 << km
        if km > 1:
            prev = k >> 1
            flip = ((idx & prev) == 0) != ((idx & k) == 0)
            a = jnp.where(flip, -a, a)
            b = jnp.where(flip, -b, b)
        for jm in range(km - 1, -1, -1):
            j = 1 << jm
            takemin = masks.get(j)
            a = _cmpx(a, j, takemin)
            b = _cmpx(b, j, takemin)
    return a, b


def _loss_kernel(t_ref, p_ref, out_ref):
    r = lax.broadcasted_iota(jnp.int32, (1, SUB, LANE), 1)
    c = lax.broadcasted_iota(jnp.int32, (1, SUB, LANE), 2)
    idx = (r >> 3) | ((r & 7) << 5) | (c << 8)
    t = t_ref[...]
    p = p_ref[...]
    d0 = t - p
    s_plain = jnp.sum(d0 * d0)
    ts, ps = _sort2_asc(t, p, idx)
    d1 = ts - ps
    s_cdf = jnp.sum(d1 * d1)
    out_ref[...] = (s_plain + s_cdf).reshape(1, 1, 1)


@jax.jit
def kernel(predictions, targets):
    rows, n = predictions.shape
    assert n == N
    t3 = targets.reshape(rows, SUB, LANE)
    p3 = predictions.reshape(rows, SUB, LANE)
    blk = 2
    partials = pl.pallas_call(
        _loss_kernel,
        grid=(rows // blk,),
        in_specs=[
            pl.BlockSpec((blk, SUB, LANE), lambda i: (i, 0, 0)),
            pl.BlockSpec((blk, SUB, LANE), lambda i: (i, 0, 0)),
        ],
        out_specs=pl.BlockSpec((1, 1, 1), lambda i: (i, 0, 0)),
        out_shape=jax.ShapeDtypeStruct((rows // blk, 1, 1), jnp.float32),
        compiler_params=pltpu.CompilerParams(
            dimension_semantics=("parallel",),
        ),
    )(t3, p3)
    total = jnp.sum(partials)
    return total / (rows * N)


# speedup vs baseline: 4.4693x; 1.0003x over previous
import functools

import jax
import jax.numpy as jnp
from jax import lax
from jax.experimental import pallas as pl
from jax.experimental.pallas import tpu as pltpu

LOG2N = 15
N = 1 << LOG2N
SUB = 256
LANE = 128


def _cmpx(x, j, takemin):
    if j < 32:
        cj = 8 * j
        g = SUB // (2 * cj)
        v = x.reshape(x.shape[0], g, 2, cj, LANE)
        a = v[:, :, 0]
        b = v[:, :, 1]
        lo = jnp.minimum(a, b)
        hi = jnp.maximum(a, b)
        out = jnp.concatenate([lo[:, :, None], hi[:, :, None]], axis=2)
        return out.reshape(x.shape[0], SUB, LANE)
    if j < SUB:
        q = j >> 5
        up = pltpu.roll(x, SUB - q, 1)
        dn = pltpu.roll(x, q, 1)
    else:
        s = j >> 8
        up = pltpu.roll(x, LANE - s, 2)
        dn = pltpu.roll(x, s, 2)
    return jnp.where(takemin, jnp.minimum(x, up), jnp.maximum(x, dn))


def _sort2_asc(a, b, idx):
    sub_iota = lax.broadcasted_iota(jnp.int32, (1, SUB, 1), 1)
    lane_iota = lax.broadcasted_iota(jnp.int32, (1, 1, LANE), 2)
    masks = {}
    for jm in range(5, LOG2N):
        j = 1 << jm
        if j < SUB:
            masks[j] = (sub_iota & (j >> 5)) == 0
        else:
            masks[j] = (lane_iota & (j >> 8)) == 0
    asc2 = (idx & 2) == 0
    a = jnp.where(asc2, a, -a)
    b = jnp.where(asc2, b, -b)
    for km in range(1, LOG2N + 1):
        k = 1 << km
        if km > 1:
            prev = k >> 1
            flip = ((idx & prev) == 0) != ((idx & k) == 0)
            a = jnp.where(flip, -a, a)
            b = jnp.where(flip, -b, b)
        for jm in range(km - 1, -1, -1):
            j = 1 << jm
            takemin = masks.get(j)
            a = _cmpx(a, j, takemin)
            b = _cmpx(b, j, takemin)
    return a, b


def _loss_kernel(t_ref, p_ref, out_ref):
    r = lax.broadcasted_iota(jnp.int32, (1, SUB, LANE), 1)
    c = lax.broadcasted_iota(jnp.int32, (1, SUB, LANE), 2)
    idx = (r >> 3) | ((r & 7) << 5) | (c << 8)
    t = t_ref[...]
    p = p_ref[...]
    d0 = t - p
    s_plain = jnp.sum(d0 * d0)
    ts, ps = _sort2_asc(t, p, idx)
    d1 = ts - ps
    s_cdf = jnp.sum(d1 * d1)
    out_ref[...] = (s_plain + s_cdf).reshape(1, 1, 1)


@jax.jit
def kernel(predictions, targets):
    rows, n = predictions.shape
    assert n == N
    t3 = targets.reshape(rows, SUB, LANE)
    p3 = predictions.reshape(rows, SUB, LANE)
    blk = 4
    partials = pl.pallas_call(
        _loss_kernel,
        grid=(rows // blk,),
        in_specs=[
            pl.BlockSpec((blk, SUB, LANE), lambda i: (i, 0, 0)),
            pl.BlockSpec((blk, SUB, LANE), lambda i: (i, 0, 0)),
        ],
        out_specs=pl.BlockSpec((1, 1, 1), lambda i: (i, 0, 0)),
        out_shape=jax.ShapeDtypeStruct((rows // blk, 1, 1), jnp.float32),
        compiler_params=pltpu.CompilerParams(
            dimension_semantics=("parallel",),
        ),
    )(t3, p3)
    total = jnp.sum(partials)
    return total / (rows * N)
